# Initial kernel scaffold; baseline (speedup 1.0000x reference)
import functools

import jax
import jax.numpy as jnp
from jax import lax
from jax.experimental import pallas as pl
from jax.experimental.pallas import tpu as pltpu
from jax.experimental.pallas import tpu_sc as plsc

N = 10000
D = 128
E = 320000

NC = 2
NS = 16
L = 16
NW = NC * NS

CH = 128
CPW = 79
EPW = CH * CPW
E_PAD = NW * EPW
N_ACC = N + 16
RPW = N // NS


def _seg_body(z_hbm, src_hbm, dst_hbm, out_hbm, sidx, didx, rows, acc, sem):
    cid = lax.axis_index("c")
    sid = lax.axis_index("s")
    wid = sid * NC + cid

    zero = jnp.zeros((L,), jnp.float32)

    def zbody(i, _):
        r = i // (D // L)
        c = i % (D // L)
        rows[r, pl.ds(c * L, L)] = zero
        return 0

    lax.fori_loop(0, CH * (D // L), zbody, 0)

    rbase = sid * RPW
    for k in range(RPW // CH):
        pltpu.sync_copy(rows, acc.at[pl.ds(rbase + k * CH, CH)])
    rem = RPW % CH
    if rem:
        pltpu.sync_copy(rows.at[pl.ds(0, rem)],
                        acc.at[pl.ds(rbase + (RPW // CH) * CH, rem)])

    plsc.subcore_barrier()

    ebase = wid * EPW

    def body(j, _):
        off = ebase + j * CH
        pltpu.sync_copy(src_hbm.at[pl.ds(off, CH)], sidx)
        pltpu.sync_copy(dst_hbm.at[pl.ds(off, CH)], didx)
        pltpu.async_copy(z_hbm.at[sidx], rows, sem).wait()
        pltpu.sync_copy(rows, acc.at[didx], add=True)
        return 0

    lax.fori_loop(0, CPW, body, 0)

    plsc.subcore_barrier()

    obase = cid * N + rbase
    for k in range(RPW // CH):
        pltpu.sync_copy(acc.at[pl.ds(rbase + k * CH, CH)],
                        out_hbm.at[pl.ds(obase + k * CH, CH)])
    if rem:
        pltpu.sync_copy(acc.at[pl.ds(rbase + (RPW // CH) * CH, rem)],
                        out_hbm.at[pl.ds(obase + (RPW // CH) * CH, rem)])


def _sc_segsum():
    return pl.kernel(
        _seg_body,
        out_type=jax.ShapeDtypeStruct((NC * N, D), jnp.float32),
        mesh=plsc.VectorSubcoreMesh(core_axis_name="c", subcore_axis_name="s",
                                    num_cores=NC, num_subcores=NS),
        scratch_types=[
            pltpu.VMEM((CH,), jnp.int32),
            pltpu.VMEM((CH,), jnp.int32),
            pltpu.VMEM((CH, D), jnp.float32),
            pltpu.VMEM_SHARED((N_ACC, D), jnp.float32),
            pltpu.SemaphoreType.DMA,
        ],
    )


def _mm_body(x_ref, w_ref, o_ref):
    o_ref[...] = jnp.dot(x_ref[...], w_ref[...],
                         preferred_element_type=jnp.float32)


def _comb_body(pa_ref, pb_ref, b_ref, w_ref, o_ref):
    h = jnp.maximum(pa_ref[...] + pb_ref[...] + b_ref[...], 0.0)
    o_ref[...] = jnp.dot(h, w_ref[...], preferred_element_type=jnp.float32)


def _comb_final_body(pa_ref, pb_ref, b_ref, w_ref, bf_ref, o_ref):
    h = jnp.maximum(pa_ref[...] + pb_ref[...] + b_ref[...], 0.0)
    o_ref[...] = jnp.dot(h, w_ref[...],
                         preferred_element_type=jnp.float32) + bf_ref[...]


_BLK = 1000
_GRID = N // _BLK


def _tc_matmul(x, w):
    return pl.pallas_call(
        _mm_body,
        grid=(_GRID,),
        in_specs=[pl.BlockSpec((_BLK, D), lambda i: (i, 0)),
                  pl.BlockSpec((D, D), lambda i: (0, 0))],
        out_specs=pl.BlockSpec((_BLK, D), lambda i: (i, 0)),
        out_shape=jax.ShapeDtypeStruct((N, D), jnp.float32),
    )(x, w)


def _tc_combine_matmul(parts, b, w):
    return pl.pallas_call(
        _comb_body,
        grid=(_GRID,),
        in_specs=[pl.BlockSpec((_BLK, D), lambda i: (i, 0)),
                  pl.BlockSpec((_BLK, D), lambda i: (i + _GRID, 0)),
                  pl.BlockSpec((1, D), lambda i: (0, 0)),
                  pl.BlockSpec((D, D), lambda i: (0, 0))],
        out_specs=pl.BlockSpec((_BLK, D), lambda i: (i, 0)),
        out_shape=jax.ShapeDtypeStruct((N, D), jnp.float32),
    )(parts, parts, b.reshape(1, D), w)


def _tc_combine_matmul_final(parts, b, w, bf):
    return pl.pallas_call(
        _comb_final_body,
        grid=(_GRID,),
        in_specs=[pl.BlockSpec((_BLK, D), lambda i: (i, 0)),
                  pl.BlockSpec((_BLK, D), lambda i: (i + _GRID, 0)),
                  pl.BlockSpec((1, D), lambda i: (0, 0)),
                  pl.BlockSpec((D, D), lambda i: (0, 0)),
                  pl.BlockSpec((1, D), lambda i: (0, 0))],
        out_specs=pl.BlockSpec((_BLK, D), lambda i: (i, 0)),
        out_shape=jax.ShapeDtypeStruct((N, D), jnp.float32),
    )(parts, parts, b.reshape(1, D), w, bf.reshape(1, D))


def kernel(features, edge_index, W1, b1, W2, b2, Wf, bf):
    src = edge_index[0].astype(jnp.int32)
    dst = edge_index[1].astype(jnp.int32)
    pad = E_PAD - E
    src = jnp.concatenate([src, jnp.zeros((pad,), jnp.int32)])
    dst = jnp.concatenate([dst, jnp.full((pad,), N, jnp.int32)])

    z1 = _tc_matmul(features, W1)
    parts1 = _sc_segsum()(z1, src, dst)
    z2 = _tc_combine_matmul(parts1, b1, W2)
    parts2 = _sc_segsum()(z2, src, dst)
    return _tc_combine_matmul_final(parts2, b2, Wf, bf)


# speedup vs baseline: 4.0900x; 4.0900x over previous
import functools

import jax
import jax.numpy as jnp
from jax import lax
from jax.experimental import pallas as pl
from jax.experimental.pallas import tpu as pltpu
from jax.experimental.pallas import tpu_sc as plsc

N = 10000
D = 128
E = 320000

NC = 2
NS = 16
L = 16
NW = NC * NS

CH = 128
CPW = 79
EPW = CH * CPW
E_PAD = NW * EPW
N_ACC = N + 16
RPW = 632


def _seg_body(z_hbm, src_hbm, dst_hbm, out_hbm, sidx, didx, rows, acc, sem):
    cid = lax.axis_index("c")
    sid = lax.axis_index("s")
    wid = sid * NC + cid

    zero = jnp.zeros((L,), jnp.float32)

    def zbody(i, _):
        r = i // (D // L)
        c = i % (D // L)
        rows[r, pl.ds(c * L, L)] = zero
        return 0

    lax.fori_loop(0, CH * (D // L), zbody, 0)

    rbase = sid * RPW
    for k in range(4):
        pltpu.sync_copy(rows, acc.at[pl.ds(rbase + k * CH, CH)])

    @pl.when(sid < NS - 1)
    def _():
        pltpu.sync_copy(rows.at[pl.ds(0, 120)],
                        acc.at[pl.ds(rbase + 4 * CH, 120)])

    @pl.when(sid == NS - 1)
    def _():
        pltpu.sync_copy(rows.at[pl.ds(0, 8)],
                        acc.at[pl.ds(rbase + 4 * CH, 8)])

    plsc.subcore_barrier()

    ebase = wid * EPW

    def body(j, _):
        off = ebase + j * CH
        pltpu.sync_copy(src_hbm.at[pl.ds(off, CH)], sidx)
        pltpu.sync_copy(dst_hbm.at[pl.ds(off, CH)], didx)
        pltpu.async_copy(z_hbm.at[sidx], rows, sem).wait()
        pltpu.sync_copy(rows, acc.at[didx], add=True)
        return 0

    lax.fori_loop(0, CPW, body, 0)

    plsc.subcore_barrier()

    for k in range(4):
        pltpu.sync_copy(acc.at[pl.ds(rbase + k * CH, CH)],
                        out_hbm.at[cid, pl.ds(rbase + k * CH, CH)])

    @pl.when(sid < NS - 1)
    def _():
        pltpu.sync_copy(acc.at[pl.ds(rbase + 4 * CH, 120)],
                        out_hbm.at[cid, pl.ds(rbase + 4 * CH, 120)])

    @pl.when(sid == NS - 1)
    def _():
        pltpu.sync_copy(acc.at[pl.ds(rbase + 4 * CH, 8)],
                        out_hbm.at[cid, pl.ds(rbase + 4 * CH, 8)])


def _sc_segsum():
    return pl.kernel(
        _seg_body,
        out_type=jax.ShapeDtypeStruct((NC, N, D), jnp.float32),
        mesh=plsc.VectorSubcoreMesh(core_axis_name="c", subcore_axis_name="s",
                                    num_cores=NC, num_subcores=NS),
        scratch_types=[
            pltpu.VMEM((CH,), jnp.int32),
            pltpu.VMEM((CH,), jnp.int32),
            pltpu.VMEM((CH, D), jnp.float32),
            pltpu.VMEM_SHARED((N_ACC, D), jnp.float32),
            pltpu.SemaphoreType.DMA,
        ],
    )


def _mm_body(x_ref, w_ref, o_ref):
    o_ref[...] = jnp.dot(x_ref[...], w_ref[...],
                         preferred_element_type=jnp.float32)


def _comb_body(pa_ref, pb_ref, b_ref, w_ref, o_ref):
    h = jnp.maximum(pa_ref[0] + pb_ref[0] + b_ref[...], 0.0)
    o_ref[...] = jnp.dot(h, w_ref[...], preferred_element_type=jnp.float32)


def _comb_final_body(pa_ref, pb_ref, b_ref, w_ref, bf_ref, o_ref):
    h = jnp.maximum(pa_ref[0] + pb_ref[0] + b_ref[...], 0.0)
    o_ref[...] = jnp.dot(h, w_ref[...],
                         preferred_element_type=jnp.float32) + bf_ref[...]


_BLK = 1000
_GRID = N // _BLK


def _tc_matmul(x, w):
    return pl.pallas_call(
        _mm_body,
        grid=(_GRID,),
        in_specs=[pl.BlockSpec((_BLK, D), lambda i: (i, 0)),
                  pl.BlockSpec((D, D), lambda i: (0, 0))],
        out_specs=pl.BlockSpec((_BLK, D), lambda i: (i, 0)),
        out_shape=jax.ShapeDtypeStruct((N, D), jnp.float32),
    )(x, w)


def _tc_combine_matmul(parts, b, w):
    return pl.pallas_call(
        _comb_body,
        grid=(_GRID,),
        in_specs=[pl.BlockSpec((1, _BLK, D), lambda i: (0, i, 0)),
                  pl.BlockSpec((1, _BLK, D), lambda i: (1, i, 0)),
                  pl.BlockSpec((1, D), lambda i: (0, 0)),
                  pl.BlockSpec((D, D), lambda i: (0, 0))],
        out_specs=pl.BlockSpec((_BLK, D), lambda i: (i, 0)),
        out_shape=jax.ShapeDtypeStruct((N, D), jnp.float32),
    )(parts, parts, b.reshape(1, D), w)


def _tc_combine_matmul_final(parts, b, w, bf):
    return pl.pallas_call(
        _comb_final_body,
        grid=(_GRID,),
        in_specs=[pl.BlockSpec((1, _BLK, D), lambda i: (0, i, 0)),
                  pl.BlockSpec((1, _BLK, D), lambda i: (1, i, 0)),
                  pl.BlockSpec((1, D), lambda i: (0, 0)),
                  pl.BlockSpec((D, D), lambda i: (0, 0)),
                  pl.BlockSpec((1, D), lambda i: (0, 0))],
        out_specs=pl.BlockSpec((_BLK, D), lambda i: (i, 0)),
        out_shape=jax.ShapeDtypeStruct((N, D), jnp.float32),
    )(parts, parts, b.reshape(1, D), w, bf.reshape(1, D))


def kernel(features, edge_index, W1, b1, W2, b2, Wf, bf):
    src = edge_index[0].astype(jnp.int32)
    dst = edge_index[1].astype(jnp.int32)
    pad = E_PAD - E
    src = jnp.concatenate([src, jnp.zeros((pad,), jnp.int32)])
    dst = jnp.concatenate([dst, jnp.full((pad,), N, jnp.int32)])

    z1 = _tc_matmul(features, W1)
    parts1 = _sc_segsum()(z1, src, dst)
    z2 = _tc_combine_matmul(parts1, b1, W2)
    parts2 = _sc_segsum()(z2, src, dst)
    return _tc_combine_matmul_final(parts2, b2, Wf, bf)
